# baseline (speedup 1.0000x reference)
import functools

import jax
import jax.numpy as jnp
from jax import lax
from jax.experimental import pallas as pl
from jax.experimental.pallas import tpu as pltpu
from jax.experimental.pallas import tpu_sc as plsc

B = 512
D = 128
L = 16
NCH = D // L
NC = 2
NS = 16
NW = NC * NS
ROWS = 128
NBUF = 6
INV = 1.0 / (B - 1)

P0 = 128
SPPT = P0 // NW
BR = 4096


def _tri_py(g):
    return g * (g - 1) // 2


def _tri(g):
    return lax.shift_right_logical(g * (g - 1), 1)


def _zeros():
    return tuple(jnp.zeros((L,), jnp.float32) for _ in range(NCH))


def _accum(buf, n, init):
    def body(r, accs):
        return tuple(a + buf[r, pl.ds(c * L, L)] for c, a in enumerate(accs))

    return lax.fori_loop(0, n, body, init)


@functools.partial(
    pl.kernel,
    mesh=plsc.VectorSubcoreMesh(core_axis_name="c", subcore_axis_name="s"),
    compiler_params=pltpu.CompilerParams(use_tc_tiling_on_sc=False),
    out_type=jax.ShapeDtypeStruct((B, D), jnp.float32),
    scratch_types=[
        pltpu.VMEM((NBUF, ROWS, D), jnp.float32),
        pltpu.VMEM((2 * SPPT, D), jnp.float32),
        pltpu.SemaphoreType.DMA,
        pltpu.SemaphoreType.DMA,
        pltpu.SemaphoreType.DMA,
        pltpu.SemaphoreType.DMA,
        pltpu.SemaphoreType.DMA,
        pltpu.SemaphoreType.DMA,
    ],
)
def _segmean_sc(emb, out, bufs, outbuf, sem0, sem1, sem2, sem3, sem4, sem5):
    sems = (sem0, sem1, sem2, sem3, sem4, sem5)
    w = lax.axis_index("s") * NC + lax.axis_index("c")
    inv = jnp.full((L,), INV, jnp.float32)

    items = []
    for j in range(SPPT):
        p = NW * j + w
        q = (B - 1) - p
        items.append((_tri(p), NW * (j + 1), p, 2 * j))
        for u in range(3):
            items.append((_tri(q) + u * ROWS, ROWS, None, None))
        items.append(
            (_tri(q) + 3 * ROWS, ROWS - 1 - NW * j, q - 3 * ROWS, 2 * j + 1)
        )

    dmas = [None] * len(items)

    def start(i):
        off, sz = items[i][0], items[i][1]
        d = pltpu.make_async_copy(
            emb.at[pl.ds(off, sz)],
            bufs.at[i % NBUF].at[pl.ds(0, sz)],
            sems[i % NBUF],
        )
        d.start()
        dmas[i] = d

    for i0 in range(NBUF - 1):
        start(i0)
    carry = None
    for i, (off, sz, nval, orow) in enumerate(items):
        dmas[i].wait()
        if i + NBUF - 1 < len(items):
            start(i + NBUF - 1)
        buf = bufs.at[i % NBUF]
        init = _zeros() if carry is None else carry
        acc = _accum(buf, sz if nval is None else nval, init)
        if orow is None:
            carry = acc
        else:
            for c in range(NCH):
                outbuf[orow, pl.ds(c * L, L)] = acc[c] * inv
            carry = None

    handles = []
    for j in range(SPPT):
        p = NW * j + w
        q = (B - 1) - p
        handles.append(pltpu.async_copy(outbuf.at[2 * j], out.at[p], sem0))
        handles.append(pltpu.async_copy(outbuf.at[2 * j + 1], out.at[q], sem1))
    for h in handles:
        h.wait()


G0, G1 = P0, B - P0
A0 = (_tri_py(G0) // BR) * BR
K0 = A0 // BR
NBLK = -(-(_tri_py(G1) - A0) // BR)

_PIECES = [[] for _ in range(NBLK)]
for _g in range(G0, G1):
    _s, _e = _tri_py(_g), _tri_py(_g) + _g
    for _kk in range((_s - A0) // BR, (_e - 1 - A0) // BR + 1):
        _blo = A0 + _kk * BR
        _PIECES[_kk].append((_g, max(_s, _blo) - _blo, min(_e, _blo + BR) - _blo))


def _tc_body(x_ref, o_ref):
    k = pl.program_id(0)

    @pl.when(k == 0)
    def _():
        o_ref[...] = jnp.zeros_like(o_ref)

    for kk, pieces in enumerate(_PIECES):
        if not pieces:
            continue

        @pl.when(k == kk)
        def _(pieces=pieces):
            for g, a, b in pieces:
                acc = None
                for c in range(a // 8, (b + 7) // 8):
                    x = x_ref[pl.ds(8 * c, 8), :]
                    if max(a, 8 * c) > 8 * c or min(b, 8 * c + 8) < 8 * c + 8:
                        rid = lax.broadcasted_iota(jnp.int32, (8, D), 0) + 8 * c
                        x = jnp.where((rid >= a) & (rid < b), x, 0.0)
                    acc = x if acc is None else acc + x
                part = jnp.sum(acc, axis=0, keepdims=True)
                o_ref[pl.ds(g - G0, 1), :] += part

    @pl.when(k == NBLK - 1)
    def _():
        o_ref[...] = o_ref[...] * INV


def kernel(graph_embedding, graph_len, weight):
    del graph_len, weight
    sc_out = _segmean_sc(graph_embedding)
    tc_out = pl.pallas_call(
        _tc_body,
        grid=(NBLK,),
        in_specs=[pl.BlockSpec((BR, D), lambda k: (k + K0, 0))],
        out_specs=pl.BlockSpec((G1 - G0, D), lambda k: (0, 0)),
        out_shape=jax.ShapeDtypeStruct((G1 - G0, D), jnp.float32),
    )(graph_embedding)
    return lax.dynamic_update_slice(sc_out, tc_out, (G0, 0))


# speedup vs baseline: 1.0191x; 1.0191x over previous
import functools

import jax
import jax.numpy as jnp
from jax import lax
from jax.experimental import pallas as pl
from jax.experimental.pallas import tpu as pltpu
from jax.experimental.pallas import tpu_sc as plsc

B = 512
D = 128
L = 16
NCH = D // L
NC = 2
NS = 16
NW = NC * NS
ROWS = 128
NBUF = 4
INV = 1.0 / (B - 1)

P0 = 128
SPPT = P0 // NW
BR = 4096


def _tri_py(g):
    return g * (g - 1) // 2


def _tri(g):
    return lax.shift_right_logical(g * (g - 1), 1)


def _zeros():
    return tuple(jnp.zeros((L,), jnp.float32) for _ in range(NCH))


def _accum(buf, n, init):
    def body(r, accs):
        return tuple(a + buf[r, pl.ds(c * L, L)] for c, a in enumerate(accs))

    return lax.fori_loop(0, n, body, init)


@functools.partial(
    pl.kernel,
    mesh=plsc.VectorSubcoreMesh(core_axis_name="c", subcore_axis_name="s"),
    compiler_params=pltpu.CompilerParams(use_tc_tiling_on_sc=False),
    out_type=jax.ShapeDtypeStruct((B, D), jnp.float32),
    scratch_types=[
        pltpu.VMEM((NBUF, ROWS, D), jnp.float32),
        pltpu.VMEM((2 * SPPT, D), jnp.float32),
        pltpu.SemaphoreType.DMA,
        pltpu.SemaphoreType.DMA,
        pltpu.SemaphoreType.DMA,
        pltpu.SemaphoreType.DMA,
    ],
)
def _segmean_sc(emb, out, bufs, outbuf, sem0, sem1, sem2, sem3):
    sems = (sem0, sem1, sem2, sem3)
    w = lax.axis_index("s") * NC + lax.axis_index("c")
    inv = jnp.full((L,), INV, jnp.float32)

    items = []
    for j in range(SPPT):
        p = NW * j + w
        q = (B - 1) - p
        items.append((_tri(p), NW * (j + 1), p, 2 * j))
        for u in range(3):
            items.append((_tri(q) + u * ROWS, ROWS, None, None))
        items.append(
            (_tri(q) + 3 * ROWS, ROWS - 1 - NW * j, q - 3 * ROWS, 2 * j + 1)
        )

    dmas = [None] * len(items)

    def start(i):
        off, sz = items[i][0], items[i][1]
        d = pltpu.make_async_copy(
            emb.at[pl.ds(off, sz)],
            bufs.at[i % NBUF].at[pl.ds(0, sz)],
            sems[i % NBUF],
        )
        d.start()
        dmas[i] = d

    for i0 in range(NBUF - 1):
        start(i0)
    carry = None
    for i, (off, sz, nval, orow) in enumerate(items):
        dmas[i].wait()
        if i + NBUF - 1 < len(items):
            start(i + NBUF - 1)
        buf = bufs.at[i % NBUF]
        init = _zeros() if carry is None else carry
        acc = _accum(buf, sz if nval is None else nval, init)
        if orow is None:
            carry = acc
        else:
            for c in range(NCH):
                outbuf[orow, pl.ds(c * L, L)] = acc[c] * inv
            carry = None

    handles = []
    for j in range(SPPT):
        p = NW * j + w
        q = (B - 1) - p
        handles.append(pltpu.async_copy(outbuf.at[2 * j], out.at[p], sem0))
        handles.append(pltpu.async_copy(outbuf.at[2 * j + 1], out.at[q], sem1))
    for h in handles:
        h.wait()


G0, G1 = P0, B - P0
A0 = (_tri_py(G0) // BR) * BR
K0 = A0 // BR
NBLK = -(-(_tri_py(G1) - A0) // BR)

_PIECES = [[] for _ in range(NBLK)]
for _g in range(G0, G1):
    _s, _e = _tri_py(_g), _tri_py(_g) + _g
    for _kk in range((_s - A0) // BR, (_e - 1 - A0) // BR + 1):
        _blo = A0 + _kk * BR
        _PIECES[_kk].append((_g, max(_s, _blo) - _blo, min(_e, _blo + BR) - _blo))


def _tc_body(x_ref, o_ref):
    k = pl.program_id(0)

    @pl.when(k == 0)
    def _():
        o_ref[...] = jnp.zeros_like(o_ref)

    for kk, pieces in enumerate(_PIECES):
        if not pieces:
            continue

        @pl.when(k == kk)
        def _(pieces=pieces):
            for g, a, b in pieces:
                acc = None
                for c in range(a // 8, (b + 7) // 8):
                    x = x_ref[pl.ds(8 * c, 8), :]
                    if max(a, 8 * c) > 8 * c or min(b, 8 * c + 8) < 8 * c + 8:
                        rid = lax.broadcasted_iota(jnp.int32, (8, D), 0) + 8 * c
                        x = jnp.where((rid >= a) & (rid < b), x, 0.0)
                    acc = x if acc is None else acc + x
                part = jnp.sum(acc, axis=0, keepdims=True)
                o_ref[pl.ds(g - G0, 1), :] += part

    @pl.when(k == NBLK - 1)
    def _():
        o_ref[...] = o_ref[...] * INV


def kernel(graph_embedding, graph_len, weight):
    del graph_len, weight
    sc_out = _segmean_sc(graph_embedding)
    tc_out = pl.pallas_call(
        _tc_body,
        grid=(NBLK,),
        in_specs=[pl.BlockSpec((BR, D), lambda k: (k + K0, 0))],
        out_specs=pl.BlockSpec((G1 - G0, D), lambda k: (0, 0)),
        out_shape=jax.ShapeDtypeStruct((G1 - G0, D), jnp.float32),
    )(graph_embedding)
    return lax.dynamic_update_slice(sc_out, tc_out, (G0, 0))
